# baseline (speedup 1.0000x reference)
import jax
import jax.numpy as jnp
from jax import lax
from jax.experimental import pallas as pl
from jax.experimental.pallas import tpu as pltpu
from jax.experimental.pallas import tpu_sc as plsc

_B = 4096
_D = 16
_ALPHA = 0.2
_NB = 10
_BR = 256
_NCHUNK = 1
_RC = _B // _NCHUNK
_NW = 32
_RPW = _RC // _NW
_RBLK = 8
_NBLK = _RPW // _RBLK
_L = 16
_NEG = -1e30


def _make_score_block(chunk):
    def _score_block(a_ref, b_ref, out_ref):
        step = pl.program_id(0) + chunk * (_RC // _BR)
        a = a_ref[...]
        b = b_ref[...]
        ha = 0.5 * jnp.sum(a * a, axis=1, keepdims=True)
        hb = 0.5 * jnp.sum(b * b, axis=1)[None, :]
        ab = lax.dot_general(a, b, (((1,), (1,)), ((), ())),
                             preferred_element_type=jnp.float32)
        m = ab - ha - hb
        row = lax.broadcasted_iota(jnp.int32, (_BR, _B), 0) + step * _BR
        col = lax.broadcasted_iota(jnp.int32, (_BR, _B), 1)
        out_ref[...] = jnp.where(row == col, _NEG, m)
    return _score_block


_WARM = 16
_CAND = (_B // _L - _WARM) * _L


def _sc_top16(m_hbm, out_hbm, buf0, buf1, cand, obuf, sem0, sem1):
    wid = lax.axis_index("s") * 2 + lax.axis_index("c")
    r0 = wid * _RPW
    bufs = (buf0, buf1)
    sems = (sem0, sem1)
    lane = lax.broadcasted_iota(jnp.int32, (_L,), 0)
    copies = [None, None]
    copies[0] = pltpu.async_copy(m_hbm.at[pl.ds(r0, _RBLK)], buf0, sem0)
    for blk in range(_NBLK):
        if blk + 1 < _NBLK:
            nxt = (blk + 1) % 2
            copies[nxt] = pltpu.async_copy(
                m_hbm.at[pl.ds(r0 + (blk + 1) * _RBLK, _RBLK)],
                bufs[nxt], sems[nxt])
        copies[blk % 2].wait()
        cur = bufs[blk % 2]

        def warm_body(c, tops):
            new = []
            for r in range(_RBLK):
                g = cur[r, pl.ds(c * _L, _L)]
                g_desc, _ = plsc.sort_key_val(g, g, descending=True)
                u = jnp.maximum(tops[r], g_desc)
                t_asc, _ = plsc.sort_key_val(u, u)
                new.append(t_asc)
            return tuple(new)

        tops = lax.fori_loop(
            0, _WARM, warm_body,
            tuple(jnp.full((_L,), _NEG, jnp.float32) for _ in range(_RBLK)))

        thr = []
        for r in range(_RBLK):
            t_s = jnp.max(jnp.where(lane <= (_L - _NB), tops[r], _NEG))
            thr.append(lax.broadcast(t_s, (_L,)))

        def filt_body(c, offs):
            offs = list(offs)
            for r in range(_RBLK):
                g = cur[r, pl.ds(c * _L, _L)]
                m = g >= thr[r]
                plsc.store_compressed(cand.at[r, pl.ds(offs[r], _L)],
                                      g, mask=m)
                offs[r] = offs[r] + jnp.sum(m.astype(jnp.int32))
            return tuple(offs)

        offs = lax.fori_loop(
            _WARM, _B // _L, filt_body,
            tuple(jnp.zeros((), jnp.int32) for _ in range(_RBLK)))

        mx = offs[0]
        for r in range(1, _RBLK):
            mx = jnp.maximum(mx, offs[r])
        trips = (mx + _L - 1) // _L
        offv = [lax.broadcast(offs[r], (_L,)) for r in range(_RBLK)]

        def merge_body(tau, tops):
            base = tau * _L
            new = []
            for r in range(_RBLK):
                g = cand[r, pl.ds(base, _L)]
                gm = jnp.where(base + lane < offv[r], g, _NEG)
                g_desc, _ = plsc.sort_key_val(gm, gm, descending=True)
                u = jnp.maximum(tops[r], g_desc)
                t_asc, _ = plsc.sort_key_val(u, u)
                new.append(t_asc)
            return tuple(new)

        tops = lax.fori_loop(0, trips, merge_body, tops)
        for r in range(_RBLK):
            obuf[r, :] = tops[r]
        pltpu.sync_copy(obuf, out_hbm.at[pl.ds(r0 + blk * _RBLK, _RBLK)])


_sc_call = pl.kernel(
    _sc_top16,
    out_type=jax.ShapeDtypeStruct((_RC, _L), jnp.float32),
    mesh=plsc.VectorSubcoreMesh(core_axis_name="c", subcore_axis_name="s"),
    scratch_types=[
        pltpu.VMEM((_RBLK, _B), jnp.float32),
        pltpu.VMEM((_RBLK, _B), jnp.float32),
        pltpu.VMEM((_RBLK, _CAND), jnp.float32),
        pltpu.VMEM((_RBLK, _L), jnp.float32),
        pltpu.SemaphoreType.DMA,
        pltpu.SemaphoreType.DMA,
    ],
    compiler_params=pltpu.CompilerParams(needs_layout_passes=False),
)


def _finish_block(sel_ref, a_ref, b_ref, out_ref):
    sel = sel_ref[...]
    a = a_ref[...]
    b = b_ref[...]
    diff = a - b
    pos2 = jnp.sum(diff * diff, axis=1, keepdims=True)
    pos = jnp.sqrt(jnp.maximum(pos2, 1e-12))
    d = jnp.sqrt(jnp.maximum(-2.0 * sel, 1e-12))
    cost = jnp.maximum(pos - d + _ALPHA, 0.0)
    lanecol = lax.broadcasted_iota(jnp.int32, (_B, _L), 1)
    kept = jnp.where(lanecol >= (_L - _NB), cost, 0.0)
    out_ref[...] = (jnp.sum(kept) * (1.0 / (_B * _NB))).reshape(1, 1)


def kernel(input1, input2, target, class1, class2):
    sels = []
    for k in range(_NCHUNK):
        m_chunk = pl.pallas_call(
            _make_score_block(k),
            grid=(_RC // _BR,),
            in_specs=[
                pl.BlockSpec((_BR, _D),
                             lambda i, k=k: (i + k * (_RC // _BR), 0)),
                pl.BlockSpec((_B, _D), lambda i: (0, 0)),
            ],
            out_specs=pl.BlockSpec((_BR, _B), lambda i: (i, 0)),
            out_shape=jax.ShapeDtypeStruct((_RC, _B), jnp.float32),
        )(input1, input2)
        sels.append(_sc_call(m_chunk))
    sel = jnp.concatenate(sels, axis=0)
    out = pl.pallas_call(
        _finish_block,
        out_shape=jax.ShapeDtypeStruct((1, 1), jnp.float32),
    )(sel, input1, input2)
    return out[0, 0]


# speedup vs baseline: 2.6634x; 2.6634x over previous
import jax
import jax.numpy as jnp
from jax import lax
from jax.experimental import pallas as pl
from jax.experimental.pallas import tpu as pltpu
from jax.experimental.pallas import tpu_sc as plsc

_B = 4096
_D = 16
_ALPHA = 0.2
_NB = 10
_BR = 256
_NW = 32
_RPW = _B // _NW
_RBLK = 8
_NBLK = _RPW // _RBLK
_L = 16
_NEG = -1e30


def _score_block(a_ref, b_ref, out_ref):
    step = pl.program_id(0)
    a = a_ref[...]
    b = b_ref[...]
    ha = 0.5 * jnp.sum(a * a, axis=1, keepdims=True)
    hb = 0.5 * jnp.sum(b * b, axis=1)[None, :]
    ab = lax.dot_general(a, b, (((1,), (1,)), ((), ())),
                         preferred_element_type=jnp.float32)
    m = ab - ha - hb
    row = lax.broadcasted_iota(jnp.int32, (_BR, _B), 0) + step * _BR
    col = lax.broadcasted_iota(jnp.int32, (_BR, _B), 1)
    m = jnp.where(row == col, _NEG, m)
    lb = lax.bitcast_convert_type(m[:, :_B // 2], jnp.int32)
    rb = lax.bitcast_convert_type(m[:, _B // 2:], jnp.int32)

    def _rnd(u):
        return u + jnp.int32(0x7FFF) + jnp.bitwise_and(
            lax.shift_right_logical(u, 16), jnp.int32(1))

    out_ref[...] = jnp.bitwise_or(
        jnp.bitwise_and(_rnd(rb), jnp.int32(-65536)),
        lax.shift_right_logical(_rnd(lb), 16))


def _sc_top16(m_hbm, out_hbm, buf0, buf1, obuf, sem0, sem1):
    wid = lax.axis_index("s") * 2 + lax.axis_index("c")
    r0 = wid * _RPW
    bufs = (buf0, buf1)
    sems = (sem0, sem1)
    copies = [None, None]
    copies[0] = pltpu.async_copy(m_hbm.at[pl.ds(r0, _RBLK)], buf0, sem0)
    for blk in range(_NBLK):
        if blk + 1 < _NBLK:
            nxt = (blk + 1) % 2
            copies[nxt] = pltpu.async_copy(
                m_hbm.at[pl.ds(r0 + (blk + 1) * _RBLK, _RBLK)],
                bufs[nxt], sems[nxt])
        copies[blk % 2].wait()
        cur = bufs[blk % 2]

        def body(c, tops):
            new = []
            for r in range(_RBLK):
                bits = cur[r, pl.ds(c * _L, _L)]
                t = tops[r]
                for half in (lax.shift_left(bits, 16),
                             jnp.bitwise_and(bits, jnp.int32(-65536))):
                    g = plsc.bitcast(half, jnp.float32)
                    g_desc, _ = plsc.sort_key_val(g, g, descending=True)
                    u = jnp.maximum(t, g_desc)
                    t, _ = plsc.sort_key_val(u, u)
                new.append(t)
            return tuple(new)

        tops = lax.fori_loop(
            0, _B // (2 * _L), body,
            tuple(jnp.full((_L,), _NEG, jnp.float32) for _ in range(_RBLK)))
        for r in range(_RBLK):
            obuf[r, :] = tops[r]
        pltpu.sync_copy(obuf, out_hbm.at[pl.ds(r0 + blk * _RBLK, _RBLK)])


_sc_call = pl.kernel(
    _sc_top16,
    out_type=jax.ShapeDtypeStruct((_B, _L), jnp.float32),
    mesh=plsc.VectorSubcoreMesh(core_axis_name="c", subcore_axis_name="s"),
    scratch_types=[
        pltpu.VMEM((_RBLK, _B // 2), jnp.int32),
        pltpu.VMEM((_RBLK, _B // 2), jnp.int32),
        pltpu.VMEM((_RBLK, _L), jnp.float32),
        pltpu.SemaphoreType.DMA,
        pltpu.SemaphoreType.DMA,
    ],
    compiler_params=pltpu.CompilerParams(needs_layout_passes=False),
)


def _finish_block(sel_ref, a_ref, b_ref, out_ref):
    sel = sel_ref[...]
    a = a_ref[...]
    b = b_ref[...]
    diff = a - b
    pos2 = jnp.sum(diff * diff, axis=1, keepdims=True)
    pos = jnp.sqrt(jnp.maximum(pos2, 1e-12))
    d = jnp.sqrt(jnp.maximum(-2.0 * sel, 1e-12))
    cost = jnp.maximum(pos - d + _ALPHA, 0.0)
    lanecol = lax.broadcasted_iota(jnp.int32, (_B, _L), 1)
    kept = jnp.where(lanecol >= (_L - _NB), cost, 0.0)
    out_ref[...] = (jnp.sum(kept) * (1.0 / (_B * _NB))).reshape(1, 1)


def kernel(input1, input2, target, class1, class2):
    m = pl.pallas_call(
        _score_block,
        grid=(_B // _BR,),
        in_specs=[
            pl.BlockSpec((_BR, _D), lambda i: (i, 0)),
            pl.BlockSpec((_B, _D), lambda i: (0, 0)),
        ],
        out_specs=pl.BlockSpec((_BR, _B // 2), lambda i: (i, 0)),
        out_shape=jax.ShapeDtypeStruct((_B, _B // 2), jnp.int32),
    )(input1, input2)
    sel = _sc_call(m)
    out = pl.pallas_call(
        _finish_block,
        out_shape=jax.ShapeDtypeStruct((1, 1), jnp.float32),
    )(sel, input1, input2)
    return out[0, 0]


# speedup vs baseline: 2.7972x; 1.0502x over previous
import jax
import jax.numpy as jnp
from jax import lax
from jax.experimental import pallas as pl
from jax.experimental.pallas import tpu as pltpu
from jax.experimental.pallas import tpu_sc as plsc

_B = 4096
_D = 16
_ALPHA = 0.2
_NB = 10
_BR = 256
_NW = 32
_RPW = _B // _NW
_RBLK = 8
_NBLK = _RPW // _RBLK
_L = 16
_NEG = -1e30


def _score_block(a_ref, b_ref, out_ref):
    step = pl.program_id(0)
    a = a_ref[...]
    b = b_ref[...]
    ha = 0.5 * jnp.sum(a * a, axis=1, keepdims=True)
    hb = 0.5 * jnp.sum(b * b, axis=1)[None, :]
    ab = lax.dot_general(a, b, (((1,), (1,)), ((), ())),
                         preferred_element_type=jnp.float32)
    m = ab - ha - hb
    row = lax.broadcasted_iota(jnp.int32, (_BR, _B), 0) + step * _BR
    col = lax.broadcasted_iota(jnp.int32, (_BR, _B), 1)
    out_ref[...] = jnp.where(row == col, _NEG, m)


def _sc_top16(m_hbm, out_hbm, buf0, buf1, obuf, sem0, sem1):
    wid = lax.axis_index("s") * 2 + lax.axis_index("c")
    r0 = wid * _RPW
    bufs = (buf0, buf1)
    sems = (sem0, sem1)
    copies = [None, None]
    copies[0] = pltpu.async_copy(m_hbm.at[pl.ds(r0, _RBLK)], buf0, sem0)
    for blk in range(_NBLK):
        if blk + 1 < _NBLK:
            nxt = (blk + 1) % 2
            copies[nxt] = pltpu.async_copy(
                m_hbm.at[pl.ds(r0 + (blk + 1) * _RBLK, _RBLK)],
                bufs[nxt], sems[nxt])
        copies[blk % 2].wait()
        cur = bufs[blk % 2]

        def body(c, tops):
            new = []
            for r in range(_RBLK):
                t_pair = []
                for h, t in zip((0, 1), tops[r]):
                    g = cur[r, pl.ds((2 * c + h) * _L, _L)]
                    g_desc, _ = plsc.sort_key_val(g, g, descending=True)
                    u = jnp.maximum(t, g_desc)
                    t_asc, _ = plsc.sort_key_val(u, u)
                    t_pair.append(t_asc)
                new.append(tuple(t_pair))
            return tuple(new)

        init = jnp.full((_L,), _NEG, jnp.float32)
        tops = lax.fori_loop(
            0, _B // (2 * _L), body,
            tuple((init, init) for _ in range(_RBLK)))
        for r in range(_RBLK):
            ta, tb = tops[r]
            tb_desc, _ = plsc.sort_key_val(tb, tb, descending=True)
            u = jnp.maximum(ta, tb_desc)
            t_fin, _ = plsc.sort_key_val(u, u)
            obuf[r, :] = t_fin
        pltpu.sync_copy(obuf, out_hbm.at[pl.ds(r0 + blk * _RBLK, _RBLK)])


_sc_call = pl.kernel(
    _sc_top16,
    out_type=jax.ShapeDtypeStruct((_B, _L), jnp.float32),
    mesh=plsc.VectorSubcoreMesh(core_axis_name="c", subcore_axis_name="s"),
    scratch_types=[
        pltpu.VMEM((_RBLK, _B), jnp.float32),
        pltpu.VMEM((_RBLK, _B), jnp.float32),
        pltpu.VMEM((_RBLK, _L), jnp.float32),
        pltpu.SemaphoreType.DMA,
        pltpu.SemaphoreType.DMA,
    ],
    compiler_params=pltpu.CompilerParams(needs_layout_passes=False),
)


def _finish_block(sel_ref, a_ref, b_ref, out_ref):
    sel = sel_ref[...]
    a = a_ref[...]
    b = b_ref[...]
    diff = a - b
    pos2 = jnp.sum(diff * diff, axis=1, keepdims=True)
    pos = jnp.sqrt(jnp.maximum(pos2, 1e-12))
    d = jnp.sqrt(jnp.maximum(-2.0 * sel, 1e-12))
    cost = jnp.maximum(pos - d + _ALPHA, 0.0)
    lanecol = lax.broadcasted_iota(jnp.int32, (_B, _L), 1)
    kept = jnp.where(lanecol >= (_L - _NB), cost, 0.0)
    out_ref[...] = (jnp.sum(kept) * (1.0 / (_B * _NB))).reshape(1, 1)


def kernel(input1, input2, target, class1, class2):
    m = pl.pallas_call(
        _score_block,
        grid=(_B // _BR,),
        in_specs=[
            pl.BlockSpec((_BR, _D), lambda i: (i, 0)),
            pl.BlockSpec((_B, _D), lambda i: (0, 0)),
        ],
        out_specs=pl.BlockSpec((_BR, _B), lambda i: (i, 0)),
        out_shape=jax.ShapeDtypeStruct((_B, _B), jnp.float32),
    )(input1, input2)
    sel = _sc_call(m)
    out = pl.pallas_call(
        _finish_block,
        out_shape=jax.ShapeDtypeStruct((1, 1), jnp.float32),
    )(sel, input1, input2)
    return out[0, 0]


# speedup vs baseline: 2.8305x; 1.0119x over previous
import jax
import jax.numpy as jnp
from jax import lax
from jax.experimental import pallas as pl
from jax.experimental.pallas import tpu as pltpu
from jax.experimental.pallas import tpu_sc as plsc

_B = 4096
_D = 16
_ALPHA = 0.2
_NB = 10
_BR = 256
_NW = 32
_RPW = _B // _NW
_RBLK = 8
_NBLK = _RPW // _RBLK
_L = 16
_NEG = -1e30


def _score_block(a_ref, b_ref, out_ref):
    step = pl.program_id(0)
    a = a_ref[...]
    b = b_ref[...]
    ha = 0.5 * jnp.sum(a * a, axis=1, keepdims=True)
    hb = 0.5 * jnp.sum(b * b, axis=1)[None, :]
    ab = lax.dot_general(a, b, (((1,), (1,)), ((), ())),
                         preferred_element_type=jnp.float32)
    m = ab - ha - hb
    row = lax.broadcasted_iota(jnp.int32, (_BR, _B), 0) + step * _BR
    col = lax.broadcasted_iota(jnp.int32, (_BR, _B), 1)
    m = jnp.where(row == col, _NEG, m)
    lb = lax.bitcast_convert_type(m[:, :_B // 2], jnp.int32)
    rb = lax.bitcast_convert_type(m[:, _B // 2:], jnp.int32)
    out_ref[...] = jnp.bitwise_or(
        jnp.bitwise_and(rb, jnp.int32(-65536)),
        lax.shift_right_logical(lb, 16))


def _sc_top16(m_hbm, out_hbm, buf0, buf1, obuf, sem0, sem1):
    wid = lax.axis_index("s") * 2 + lax.axis_index("c")
    r0 = wid * _RPW
    bufs = (buf0, buf1)
    sems = (sem0, sem1)
    copies = [None, None]
    copies[0] = pltpu.async_copy(m_hbm.at[pl.ds(r0, _RBLK)], buf0, sem0)
    for blk in range(_NBLK):
        if blk + 1 < _NBLK:
            nxt = (blk + 1) % 2
            copies[nxt] = pltpu.async_copy(
                m_hbm.at[pl.ds(r0 + (blk + 1) * _RBLK, _RBLK)],
                bufs[nxt], sems[nxt])
        copies[blk % 2].wait()
        cur = bufs[blk % 2]

        def body(c, tops):
            new = []
            for r in range(_RBLK):
                bits = cur[r, pl.ds(c * _L, _L)]
                t_pair = []
                for half, t in zip((lax.shift_left(bits, 16),
                                    jnp.bitwise_and(bits, jnp.int32(-65536))),
                                   tops[r]):
                    g = plsc.bitcast(half, jnp.float32)
                    g_desc, _ = plsc.sort_key_val(g, g, descending=True)
                    u = jnp.maximum(t, g_desc)
                    t_asc, _ = plsc.sort_key_val(u, u)
                    t_pair.append(t_asc)
                new.append(tuple(t_pair))
            return tuple(new)

        init = jnp.full((_L,), _NEG, jnp.float32)
        tops = lax.fori_loop(
            0, _B // (2 * _L), body,
            tuple((init, init) for _ in range(_RBLK)))
        for r in range(_RBLK):
            ta, tb = tops[r]
            tb_desc, _ = plsc.sort_key_val(tb, tb, descending=True)
            u = jnp.maximum(ta, tb_desc)
            t_fin, _ = plsc.sort_key_val(u, u)
            obuf[r, :] = t_fin
        pltpu.sync_copy(obuf, out_hbm.at[pl.ds(r0 + blk * _RBLK, _RBLK)])


_sc_call = pl.kernel(
    _sc_top16,
    out_type=jax.ShapeDtypeStruct((_B, _L), jnp.float32),
    mesh=plsc.VectorSubcoreMesh(core_axis_name="c", subcore_axis_name="s"),
    scratch_types=[
        pltpu.VMEM((_RBLK, _B // 2), jnp.int32),
        pltpu.VMEM((_RBLK, _B // 2), jnp.int32),
        pltpu.VMEM((_RBLK, _L), jnp.float32),
        pltpu.SemaphoreType.DMA,
        pltpu.SemaphoreType.DMA,
    ],
    compiler_params=pltpu.CompilerParams(needs_layout_passes=False),
)


def _finish_block(sel_ref, a_ref, b_ref, out_ref):
    sel = sel_ref[...]
    a = a_ref[...]
    b = b_ref[...]
    diff = a - b
    pos2 = jnp.sum(diff * diff, axis=1, keepdims=True)
    pos = jnp.sqrt(jnp.maximum(pos2, 1e-12))
    d = jnp.sqrt(jnp.maximum(-2.0 * sel, 1e-12))
    cost = jnp.maximum(pos - d + _ALPHA, 0.0)
    lanecol = lax.broadcasted_iota(jnp.int32, (_B, _L), 1)
    kept = jnp.where(lanecol >= (_L - _NB), cost, 0.0)
    out_ref[...] = (jnp.sum(kept) * (1.0 / (_B * _NB))).reshape(1, 1)


def kernel(input1, input2, target, class1, class2):
    m = pl.pallas_call(
        _score_block,
        grid=(_B // _BR,),
        in_specs=[
            pl.BlockSpec((_BR, _D), lambda i: (i, 0)),
            pl.BlockSpec((_B, _D), lambda i: (0, 0)),
        ],
        out_specs=pl.BlockSpec((_BR, _B // 2), lambda i: (i, 0)),
        out_shape=jax.ShapeDtypeStruct((_B, _B // 2), jnp.int32),
    )(input1, input2)
    sel = _sc_call(m)
    out = pl.pallas_call(
        _finish_block,
        out_shape=jax.ShapeDtypeStruct((1, 1), jnp.float32),
    )(sel, input1, input2)
    return out[0, 0]


# speedup vs baseline: 3.0066x; 1.0622x over previous
import jax
import jax.numpy as jnp
from jax import lax
from jax.experimental import pallas as pl
from jax.experimental.pallas import tpu as pltpu
from jax.experimental.pallas import tpu_sc as plsc

_B = 4096
_D = 16
_ALPHA = 0.2
_NB = 10
_BR = 512
_NW = 32
_RPW = _B // _NW
_RBLK = 8
_NBLK = _RPW // _RBLK
_L = 16
_NEG = -1e30


def _score_block(a_ref, b_ref, out_ref):
    step = pl.program_id(0)
    a = a_ref[...]
    b = b_ref[...]
    ha = 0.5 * jnp.sum(a * a, axis=1, keepdims=True)
    hb = 0.5 * jnp.sum(b * b, axis=1)[None, :]
    ab = lax.dot_general(a, b, (((1,), (1,)), ((), ())),
                         preferred_element_type=jnp.float32)
    m = ab - ha - hb
    row = lax.broadcasted_iota(jnp.int32, (_BR, _B), 0) + step * _BR
    col = lax.broadcasted_iota(jnp.int32, (_BR, _B), 1)
    out_ref[...] = jnp.where(row == col, _NEG, m)


def _sc_top16(m_hbm, out_hbm, buf0, buf1, obuf, sem0, sem1):
    wid = lax.axis_index("s") * 2 + lax.axis_index("c")
    r0 = wid * _RPW
    bufs = (buf0, buf1)
    sems = (sem0, sem1)
    copies = [None, None]
    copies[0] = pltpu.async_copy(m_hbm.at[pl.ds(r0, _RBLK)], buf0, sem0)
    for blk in range(_NBLK):
        if blk + 1 < _NBLK:
            nxt = (blk + 1) % 2
            copies[nxt] = pltpu.async_copy(
                m_hbm.at[pl.ds(r0 + (blk + 1) * _RBLK, _RBLK)],
                bufs[nxt], sems[nxt])
        copies[blk % 2].wait()
        cur = bufs[blk % 2]

        def body(c, tops):
            new = []
            for r in range(_RBLK):
                g = cur[r, pl.ds(c * _L, _L)]
                g_desc, _ = plsc.sort_key_val(g, g, descending=True)
                u = jnp.maximum(tops[r], g_desc)
                t_asc, _ = plsc.sort_key_val(u, u)
                new.append(t_asc)
            return tuple(new)

        tops = lax.fori_loop(
            0, _B // _L, body,
            tuple(jnp.full((_L,), _NEG, jnp.float32) for _ in range(_RBLK)))
        for r in range(_RBLK):
            obuf[r, :] = tops[r]
        pltpu.sync_copy(obuf, out_hbm.at[pl.ds(r0 + blk * _RBLK, _RBLK)])


_sc_call = pl.kernel(
    _sc_top16,
    out_type=jax.ShapeDtypeStruct((_B, _L), jnp.float32),
    mesh=plsc.VectorSubcoreMesh(core_axis_name="c", subcore_axis_name="s"),
    scratch_types=[
        pltpu.VMEM((_RBLK, _B), jnp.float32),
        pltpu.VMEM((_RBLK, _B), jnp.float32),
        pltpu.VMEM((_RBLK, _L), jnp.float32),
        pltpu.SemaphoreType.DMA,
        pltpu.SemaphoreType.DMA,
    ],
    compiler_params=pltpu.CompilerParams(needs_layout_passes=False),
)


def _finish_block(sel_ref, a_ref, b_ref, out_ref):
    sel = sel_ref[...]
    a = a_ref[...]
    b = b_ref[...]
    diff = a - b
    pos2 = jnp.sum(diff * diff, axis=1, keepdims=True)
    pos = jnp.sqrt(jnp.maximum(pos2, 1e-12))
    d = jnp.sqrt(jnp.maximum(-2.0 * sel, 1e-12))
    cost = jnp.maximum(pos - d + _ALPHA, 0.0)
    lanecol = lax.broadcasted_iota(jnp.int32, (_B, _L), 1)
    kept = jnp.where(lanecol >= (_L - _NB), cost, 0.0)
    out_ref[...] = (jnp.sum(kept) * (1.0 / (_B * _NB))).reshape(1, 1)


def kernel(input1, input2, target, class1, class2):
    m = pl.pallas_call(
        _score_block,
        grid=(_B // _BR,),
        in_specs=[
            pl.BlockSpec((_BR, _D), lambda i: (i, 0)),
            pl.BlockSpec((_B, _D), lambda i: (0, 0)),
        ],
        out_specs=pl.BlockSpec((_BR, _B), lambda i: (i, 0)),
        out_shape=jax.ShapeDtypeStruct((_B, _B), jnp.float32),
    )(input1, input2)
    sel = _sc_call(m)
    out = pl.pallas_call(
        _finish_block,
        out_shape=jax.ShapeDtypeStruct((1, 1), jnp.float32),
    )(sel, input1, input2)
    return out[0, 0]
